# Initial kernel scaffold; baseline (speedup 1.0000x reference)
import jax
import jax.numpy as jnp
from jax.experimental import pallas as pl


def kernel(inputs, hidden_state, Wg0, bg0, Wc0, bc0, Wg, bg, Wc, bc, Wp, bp, support):
    raise NotImplementedError("write your pallas kernel here")


# speedup vs baseline: 3.4827x; 3.4827x over previous
import jax
import jax.numpy as jnp
from jax.experimental import pallas as pl
from jax.experimental.pallas import tpu as pltpu

N = 512
B = 32
U = 64
L = 4
NM = 3
IN0 = N + U
INL = 2 * U


def _dot(a, b):
    return jax.lax.dot_general(a, b, (((1,), (0,)), ((), ())),
                               preferred_element_type=jnp.float32)


def _decoder_kernel(x_ref, h_ref, wg0_ref, bg0_ref, wc0_ref, bc0_ref,
                    wg_ref, bg_ref, wc_ref, bc_ref, wp_ref, bp_ref, s_ref,
                    out_ref, hs_ref):
    S = s_ref[...]

    def gconv(x0, w, b):
        p0 = _dot(x0, w[0])
        p1 = _dot(x0, w[1])
        p2 = _dot(x0, w[2])
        return p0 - p2 + _dot(S, p1 + 2.0 * _dot(S, p2)) + b

    def cell(xi, h, wgs, bgv, wcs, bcv):
        val = jax.nn.sigmoid(gconv(jnp.concatenate([xi, h], axis=1), wgs, bgv))
        r = val[:, :U]
        u = val[:, U:]
        c = jnp.tanh(gconv(jnp.concatenate([xi, r * h], axis=1), wcs, bcv))
        return u * h + (1.0 - u) * c

    h = cell(x_ref[0], h_ref[0, 0], wg0_ref[...], bg0_ref[...],
             wc0_ref[...], bc0_ref[...])
    hs_ref[0, 0] = h
    for l in range(L - 1):
        h = cell(h, h_ref[l + 1, 0], wg_ref[l], bg_ref[l], wc_ref[l], bc_ref[l])
        hs_ref[l + 1, 0] = h
    out_ref[0] = _dot(h, wp_ref[...]) + bp_ref[...]


def kernel(inputs, hidden_state, Wg0, bg0, Wc0, bc0, Wg, bg, Wc, bc, Wp, bp, support):
    x = inputs.reshape(B, N, N)
    h0 = hidden_state.reshape(L, B, N, U)
    wg0 = Wg0.reshape(IN0, NM, 2 * U).transpose(1, 0, 2)
    wc0 = Wc0.reshape(IN0, NM, U).transpose(1, 0, 2)
    wg = Wg.reshape(L - 1, INL, NM, 2 * U).transpose(0, 2, 1, 3)
    wc = Wc.reshape(L - 1, INL, NM, U).transpose(0, 2, 1, 3)
    bg0r = bg0.reshape(1, 2 * U)
    bc0r = bc0.reshape(1, U)
    bgr = bg.reshape(L - 1, 1, 2 * U)
    bcr = bc.reshape(L - 1, 1, U)
    bpr = bp.reshape(1, N)

    out, hs = pl.pallas_call(
        _decoder_kernel,
        grid=(B,),
        in_specs=[
            pl.BlockSpec((1, N, N), lambda b: (b, 0, 0)),
            pl.BlockSpec((L, 1, N, U), lambda b: (0, b, 0, 0)),
            pl.BlockSpec((NM, IN0, 2 * U), lambda b: (0, 0, 0)),
            pl.BlockSpec((1, 2 * U), lambda b: (0, 0)),
            pl.BlockSpec((NM, IN0, U), lambda b: (0, 0, 0)),
            pl.BlockSpec((1, U), lambda b: (0, 0)),
            pl.BlockSpec((L - 1, NM, INL, 2 * U), lambda b: (0, 0, 0, 0)),
            pl.BlockSpec((L - 1, 1, 2 * U), lambda b: (0, 0, 0)),
            pl.BlockSpec((L - 1, NM, INL, U), lambda b: (0, 0, 0, 0)),
            pl.BlockSpec((L - 1, 1, U), lambda b: (0, 0, 0)),
            pl.BlockSpec((U, N), lambda b: (0, 0)),
            pl.BlockSpec((1, N), lambda b: (0, 0)),
            pl.BlockSpec((N, N), lambda b: (0, 0)),
        ],
        out_specs=[
            pl.BlockSpec((1, N, N), lambda b: (b, 0, 0)),
            pl.BlockSpec((L, 1, N, U), lambda b: (0, b, 0, 0)),
        ],
        out_shape=[
            jax.ShapeDtypeStruct((B, N, N), jnp.float32),
            jax.ShapeDtypeStruct((L, B, N, U), jnp.float32),
        ],
        compiler_params=pltpu.CompilerParams(
            dimension_semantics=("parallel",),
        ),
    )(x, h0, wg0, bg0r, wc0, bc0r, wg, bgr, wc, bcr, Wp, bpr, support)
    return out.reshape(B, N * N), hs.reshape(L, B, N * U)


# speedup vs baseline: 3.5006x; 1.0052x over previous
import jax
import jax.numpy as jnp
from jax.experimental import pallas as pl
from jax.experimental.pallas import tpu as pltpu

N = 512
B = 32
U = 64
L = 4
NM = 3
IN0 = N + U
INL = 2 * U
GB = 4

_BF16 = jnp.bfloat16


def _dot(a, b):
    return jax.lax.dot_general(a, b, (((1,), (0,)), ((), ())),
                               preferred_element_type=jnp.float32)


def _decoder_kernel(x_ref, h_ref, wg0_ref, bg0_ref, wc0_ref, bc0_ref,
                    wg_ref, bg_ref, wc_ref, bc_ref, wp_ref, bp_ref, s_ref,
                    out_ref, hs_ref):
    S = s_ref[...]

    def gconv(x0, w, b):
        p0 = _dot(x0, w[0])
        p1 = _dot(x0, w[1])
        p2 = _dot(x0, w[2])
        sp2 = _dot(S, p2.astype(_BF16))
        return p0 - p2 + _dot(S, (p1 + 2.0 * sp2).astype(_BF16)) + b

    def cell(xi, h, wgs, bgv, wcs, bcv):
        val = jax.nn.sigmoid(gconv(
            jnp.concatenate([xi, h.astype(_BF16)], axis=1), wgs, bgv))
        r = val[:, :U]
        u = val[:, U:]
        c = jnp.tanh(gconv(
            jnp.concatenate([xi, (r * h).astype(_BF16)], axis=1), wcs, bcv))
        return u * h + (1.0 - u) * c

    for g in range(GB):
        h = cell(x_ref[g].astype(_BF16), h_ref[0, g], wg0_ref[...],
                 bg0_ref[...], wc0_ref[...], bc0_ref[...])
        hs_ref[0, g] = h
        for l in range(L - 1):
            h = cell(h.astype(_BF16), h_ref[l + 1, g],
                     wg_ref[l], bg_ref[l], wc_ref[l], bc_ref[l])
            hs_ref[l + 1, g] = h
        out_ref[g] = _dot(h.astype(_BF16), wp_ref[...]) + bp_ref[...]


def kernel(inputs, hidden_state, Wg0, bg0, Wc0, bc0, Wg, bg, Wc, bc, Wp, bp, support):
    x = inputs.reshape(B, N, N)
    h0 = hidden_state.reshape(L, B, N, U)
    wg0 = Wg0.reshape(IN0, NM, 2 * U).transpose(1, 0, 2).astype(_BF16)
    wc0 = Wc0.reshape(IN0, NM, U).transpose(1, 0, 2).astype(_BF16)
    wg = Wg.reshape(L - 1, INL, NM, 2 * U).transpose(0, 2, 1, 3).astype(_BF16)
    wc = Wc.reshape(L - 1, INL, NM, U).transpose(0, 2, 1, 3).astype(_BF16)
    bg0r = bg0.reshape(1, 2 * U)
    bc0r = bc0.reshape(1, U)
    bgr = bg.reshape(L - 1, 1, 2 * U)
    bcr = bc.reshape(L - 1, 1, U)
    bpr = bp.reshape(1, N)

    out, hs = pl.pallas_call(
        _decoder_kernel,
        grid=(B // GB,),
        in_specs=[
            pl.BlockSpec((GB, N, N), lambda g: (g, 0, 0)),
            pl.BlockSpec((L, GB, N, U), lambda g: (0, g, 0, 0)),
            pl.BlockSpec((NM, IN0, 2 * U), lambda g: (0, 0, 0)),
            pl.BlockSpec((1, 2 * U), lambda g: (0, 0)),
            pl.BlockSpec((NM, IN0, U), lambda g: (0, 0, 0)),
            pl.BlockSpec((1, U), lambda g: (0, 0)),
            pl.BlockSpec((L - 1, NM, INL, 2 * U), lambda g: (0, 0, 0, 0)),
            pl.BlockSpec((L - 1, 1, 2 * U), lambda g: (0, 0, 0)),
            pl.BlockSpec((L - 1, NM, INL, U), lambda g: (0, 0, 0, 0)),
            pl.BlockSpec((L - 1, 1, U), lambda g: (0, 0, 0)),
            pl.BlockSpec((U, N), lambda g: (0, 0)),
            pl.BlockSpec((1, N), lambda g: (0, 0)),
            pl.BlockSpec((N, N), lambda g: (0, 0)),
        ],
        out_specs=[
            pl.BlockSpec((GB, N, N), lambda g: (g, 0, 0)),
            pl.BlockSpec((L, GB, N, U), lambda g: (0, g, 0, 0)),
        ],
        out_shape=[
            jax.ShapeDtypeStruct((B, N, N), jnp.float32),
            jax.ShapeDtypeStruct((L, B, N, U), jnp.float32),
        ],
        compiler_params=pltpu.CompilerParams(
            dimension_semantics=("parallel",),
        ),
    )(x, h0, wg0, bg0r, wc0, bc0r, wg, bgr, wc, bcr,
      Wp.astype(_BF16), bpr, support.astype(_BF16))
    return out.reshape(B, N * N), hs.reshape(L, B, N * U)


# speedup vs baseline: 4.7041x; 1.3438x over previous
import jax
import jax.numpy as jnp
from jax.experimental import pallas as pl
from jax.experimental.pallas import tpu as pltpu

N = 512
B = 32
U = 64
L = 4
NM = 3
IN0 = N + U
INL = 2 * U
GB = 4

_BF16 = jnp.bfloat16


def _dot(a, b):
    return jax.lax.dot_general(a, b, (((1,), (0,)), ((), ())),
                               preferred_element_type=jnp.float32)


def _decoder_kernel(x_ref, h_ref, wg0x_ref, wg0h_ref, bg0_ref,
                    wc0x_ref, wc0h_ref, bc0_ref,
                    wgx_ref, wgh_ref, bg_ref, wcx_ref, wch_ref, bc_ref,
                    wp_ref, bp_ref, s_ref, out_ref, hs_ref):
    S = s_ref[...]

    def hop(p0, p1, p2, b):
        p13 = p1.reshape(GB, N, -1)
        p23 = p2.reshape(GB, N, -1)
        ys = []
        for g in range(GB):
            sp2 = _dot(S, p23[g].astype(_BF16))
            ys.append(_dot(S, (p13[g] + 2.0 * sp2).astype(_BF16)))
        sy = jnp.stack(ys, axis=0).reshape(p0.shape)
        return p0 - p2 + sy + b

    def gconv(xi, st, wx, wh, b):
        p0 = _dot(xi, wx[0]) + _dot(st, wh[0])
        p1 = _dot(xi, wx[1]) + _dot(st, wh[1])
        p2 = _dot(xi, wx[2]) + _dot(st, wh[2])
        return hop(p0, p1, p2, b)

    def cell(xi, h, wx, wh, bgv, wcx, wch, bcv):
        val = jax.nn.sigmoid(gconv(xi, h.astype(_BF16), wx, wh, bgv))
        r = val[:, :U]
        u = val[:, U:]
        c = jnp.tanh(gconv(xi, (r * h).astype(_BF16), wcx, wch, bcv))
        return u * h + (1.0 - u) * c

    xi = x_ref[...].reshape(GB * N, N).astype(_BF16)
    h = cell(xi, h_ref[0].reshape(GB * N, U),
             wg0x_ref[...], wg0h_ref[...], bg0_ref[...],
             wc0x_ref[...], wc0h_ref[...], bc0_ref[...])
    hs_ref[0] = h.reshape(GB, N, U)
    for l in range(L - 1):
        h = cell(h.astype(_BF16), h_ref[l + 1].reshape(GB * N, U),
                 wgx_ref[l], wgh_ref[l], bg_ref[l],
                 wcx_ref[l], wch_ref[l], bc_ref[l])
        hs_ref[l + 1] = h.reshape(GB, N, U)
    proj = _dot(h.astype(_BF16), wp_ref[...]) + bp_ref[...]
    out_ref[...] = proj.reshape(GB, N, N)


def kernel(inputs, hidden_state, Wg0, bg0, Wc0, bc0, Wg, bg, Wc, bc, Wp, bp, support):
    x = inputs.reshape(B, N, N)
    h0 = hidden_state.reshape(L, B, N, U)
    wg0 = Wg0.reshape(IN0, NM, 2 * U).transpose(1, 0, 2).astype(_BF16)
    wg0x, wg0h = wg0[:, :N, :], wg0[:, N:, :]
    wc0 = Wc0.reshape(IN0, NM, U).transpose(1, 0, 2).astype(_BF16)
    wc0x, wc0h = wc0[:, :N, :], wc0[:, N:, :]
    wg = Wg.reshape(L - 1, INL, NM, 2 * U).transpose(0, 2, 1, 3).astype(_BF16)
    wgx, wgh = wg[:, :, :U, :], wg[:, :, U:, :]
    wc = Wc.reshape(L - 1, INL, NM, U).transpose(0, 2, 1, 3).astype(_BF16)
    wcx, wch = wc[:, :, :U, :], wc[:, :, U:, :]
    bg0r = bg0.reshape(1, 2 * U)
    bc0r = bc0.reshape(1, U)
    bgr = bg.reshape(L - 1, 1, 2 * U)
    bcr = bc.reshape(L - 1, 1, U)
    bpr = bp.reshape(1, N)

    out, hs = pl.pallas_call(
        _decoder_kernel,
        grid=(B // GB,),
        in_specs=[
            pl.BlockSpec((GB, N, N), lambda g: (g, 0, 0)),
            pl.BlockSpec((L, GB, N, U), lambda g: (0, g, 0, 0)),
            pl.BlockSpec((NM, N, 2 * U), lambda g: (0, 0, 0)),
            pl.BlockSpec((NM, U, 2 * U), lambda g: (0, 0, 0)),
            pl.BlockSpec((1, 2 * U), lambda g: (0, 0)),
            pl.BlockSpec((NM, N, U), lambda g: (0, 0, 0)),
            pl.BlockSpec((NM, U, U), lambda g: (0, 0, 0)),
            pl.BlockSpec((1, U), lambda g: (0, 0)),
            pl.BlockSpec((L - 1, NM, U, 2 * U), lambda g: (0, 0, 0, 0)),
            pl.BlockSpec((L - 1, NM, U, 2 * U), lambda g: (0, 0, 0, 0)),
            pl.BlockSpec((L - 1, 1, 2 * U), lambda g: (0, 0, 0)),
            pl.BlockSpec((L - 1, NM, U, U), lambda g: (0, 0, 0, 0)),
            pl.BlockSpec((L - 1, NM, U, U), lambda g: (0, 0, 0, 0)),
            pl.BlockSpec((L - 1, 1, U), lambda g: (0, 0, 0)),
            pl.BlockSpec((U, N), lambda g: (0, 0)),
            pl.BlockSpec((1, N), lambda g: (0, 0)),
            pl.BlockSpec((N, N), lambda g: (0, 0)),
        ],
        out_specs=[
            pl.BlockSpec((GB, N, N), lambda g: (g, 0, 0)),
            pl.BlockSpec((L, GB, N, U), lambda g: (0, g, 0, 0)),
        ],
        out_shape=[
            jax.ShapeDtypeStruct((B, N, N), jnp.float32),
            jax.ShapeDtypeStruct((L, B, N, U), jnp.float32),
        ],
        compiler_params=pltpu.CompilerParams(
            dimension_semantics=("parallel",),
        ),
    )(x, h0, wg0x, wg0h, bg0r, wc0x, wc0h, bc0r,
      wgx, wgh, bgr, wcx, wch, bcr,
      Wp.astype(_BF16), bpr, support.astype(_BF16))
    return out.reshape(B, N * N), hs.reshape(L, B, N * U)


# speedup vs baseline: 5.3189x; 1.1307x over previous
import jax
import jax.numpy as jnp
from jax.experimental import pallas as pl
from jax.experimental.pallas import tpu as pltpu

N = 512
B = 32
U = 64
L = 4
NM = 3
IN0 = N + U
INL = 2 * U
GB = 4
GBN = GB * N

_BF16 = jnp.bfloat16


def _dot(a, b):
    return jax.lax.dot_general(a, b, (((1,), (0,)), ((), ())),
                               preferred_element_type=jnp.float32)


def _decoder_kernel(x_ref, h_ref, wx0_ref, whg0_ref, whc0_ref,
                    wx_ref, whg_ref, whc_ref, bg_ref, bc_ref,
                    wp_ref, bp_ref, s_ref, out_ref, hs_ref, t2_ref):
    S = s_ref[...]

    @pl.when(pl.program_id(0) == 0)
    def _():
        ii = jax.lax.broadcasted_iota(jnp.int32, (N, N), 0)
        jj = jax.lax.broadcasted_iota(jnp.int32, (N, N), 1)
        eye = jnp.where(ii == jj, 1.0, 0.0)
        t2_ref[...] = (2.0 * _dot(S, S) - eye).astype(_BF16)

    T2 = t2_ref[...]

    def hops(pb, lo, out):
        p3 = pb.reshape(GB, N, -1)
        ys = [_dot(S, p3[g][:, lo:lo + out]) +
              _dot(T2, p3[g][:, lo + out:lo + 2 * out]) for g in range(GB)]
        return jnp.stack(ys, axis=0).reshape(GBN, out)

    def cell(xi, h, wx, whg, whc, bgv, bcv):
        xp = _dot(xi, wx)
        pg = xp[:, :384] + _dot(h.astype(_BF16), whg)
        pgb = pg.astype(_BF16)
        val = jax.nn.sigmoid(pg[:, :128] + hops(pgb, 128, 128) + bgv)
        r = val[:, :U]
        u = val[:, U:]
        pc = xp[:, 384:] + _dot((r * h).astype(_BF16), whc)
        pcb = pc.astype(_BF16)
        c = jnp.tanh(pc[:, 128:] + hops(pcb, 0, U) + bcv)
        return u * h + (1.0 - u) * c

    xi = x_ref[...].reshape(GBN, N).astype(_BF16)
    h = cell(xi, h_ref[0].reshape(GBN, U), wx0_ref[...], whg0_ref[...],
             whc0_ref[...], bg_ref[0], bc_ref[0])
    hs_ref[0] = h.reshape(GB, N, U)
    for l in range(L - 1):
        h = cell(h.astype(_BF16), h_ref[l + 1].reshape(GBN, U),
                 wx_ref[l], whg_ref[l], whc_ref[l],
                 bg_ref[l + 1], bc_ref[l + 1])
        hs_ref[l + 1] = h.reshape(GB, N, U)
    proj = _dot(h.astype(_BF16), wp_ref[...]) + bp_ref[...]
    out_ref[...] = proj.reshape(GB, N, N)


def _merge_weights(Wgl, Wcl, d):
    wg = Wgl.reshape(d + U, NM, 2 * U).transpose(1, 0, 2)
    wc = Wcl.reshape(d + U, NM, U).transpose(1, 0, 2)
    wx = jnp.concatenate([wg[0, :d], wg[1, :d], wg[2, :d],
                          wc[1, :d], wc[2, :d], wc[0, :d]], axis=1)
    whg = jnp.concatenate([wg[0, d:], wg[1, d:], wg[2, d:]], axis=1)
    whc = jnp.concatenate([wc[1, d:], wc[2, d:], wc[0, d:]], axis=1)
    return wx, whg, whc


def kernel(inputs, hidden_state, Wg0, bg0, Wc0, bc0, Wg, bg, Wc, bc, Wp, bp, support):
    x = inputs.reshape(B, N, N)
    h0 = hidden_state.reshape(L, B, N, U)
    wx0, whg0, whc0 = _merge_weights(Wg0, Wc0, N)
    mw = [_merge_weights(Wg[l], Wc[l], U) for l in range(L - 1)]
    wx = jnp.stack([m[0] for m in mw]).astype(_BF16)
    whg = jnp.stack([m[1] for m in mw]).astype(_BF16)
    whc = jnp.stack([m[2] for m in mw]).astype(_BF16)
    bgall = jnp.concatenate([bg0.reshape(1, 1, 2 * U),
                             bg.reshape(L - 1, 1, 2 * U)], axis=0)
    bcall = jnp.concatenate([bc0.reshape(1, 1, U),
                             bc.reshape(L - 1, 1, U)], axis=0)
    bpr = bp.reshape(1, N)

    out, hs = pl.pallas_call(
        _decoder_kernel,
        grid=(B // GB,),
        in_specs=[
            pl.BlockSpec((GB, N, N), lambda g: (g, 0, 0)),
            pl.BlockSpec((L, GB, N, U), lambda g: (0, g, 0, 0)),
            pl.BlockSpec((N, 9 * U), lambda g: (0, 0)),
            pl.BlockSpec((U, 6 * U), lambda g: (0, 0)),
            pl.BlockSpec((U, 3 * U), lambda g: (0, 0)),
            pl.BlockSpec((L - 1, U, 9 * U), lambda g: (0, 0, 0)),
            pl.BlockSpec((L - 1, U, 6 * U), lambda g: (0, 0, 0)),
            pl.BlockSpec((L - 1, U, 3 * U), lambda g: (0, 0, 0)),
            pl.BlockSpec((L, 1, 2 * U), lambda g: (0, 0, 0)),
            pl.BlockSpec((L, 1, U), lambda g: (0, 0, 0)),
            pl.BlockSpec((U, N), lambda g: (0, 0)),
            pl.BlockSpec((1, N), lambda g: (0, 0)),
            pl.BlockSpec((N, N), lambda g: (0, 0)),
        ],
        out_specs=[
            pl.BlockSpec((GB, N, N), lambda g: (g, 0, 0)),
            pl.BlockSpec((L, GB, N, U), lambda g: (0, g, 0, 0)),
        ],
        out_shape=[
            jax.ShapeDtypeStruct((B, N, N), jnp.float32),
            jax.ShapeDtypeStruct((L, B, N, U), jnp.float32),
        ],
        scratch_shapes=[pltpu.VMEM((N, N), _BF16)],
        compiler_params=pltpu.CompilerParams(
            dimension_semantics=("parallel",),
        ),
    )(x, h0, wx0.astype(_BF16), whg0.astype(_BF16), whc0.astype(_BF16),
      wx, whg, whc, bgall, bcall,
      Wp.astype(_BF16), bpr, support.astype(_BF16))
    return out.reshape(B, N * N), hs.reshape(L, B, N * U)
